# baseline (speedup 1.0000x reference)
import functools

import jax
import jax.numpy as jnp
from jax import lax
from jax.experimental import pallas as pl
from jax.experimental.pallas import tpu as pltpu
from jax.experimental.pallas import tpu_sc as plsc

N_NODES = 10000
N_EDGES = 320000
D = 128

NC = 2
NS = 16
LANES = 16

B = 128
KP = 160
K0 = 128
K1 = KP - K0
HC = 8
NHC = KP // HC
E_PAD = NS * KP * B
N_PAD = 10240
ROWS_PER_TILE = N_PAD // NS


def _sc_segment_sum(x, src3, dst3):
    mesh = plsc.VectorSubcoreMesh(core_axis_name="c", subcore_axis_name="s")

    @functools.partial(
        pl.kernel,
        out_type=(
            jax.ShapeDtypeStruct((NC, N_PAD, D), jnp.float32),
            jax.ShapeDtypeStruct((NC, NS, N_PAD), jnp.float32),
        ),
        mesh=mesh,
        scratch_types=[
            pltpu.VMEM((2 * HC, B), jnp.int32),
            pltpu.VMEM((2 * HC, B), jnp.int32),
            pltpu.VMEM((B, D), jnp.float32),
            pltpu.VMEM((B, D), jnp.float32),
            pltpu.VMEM((N_PAD,), jnp.float32),
            pltpu.VMEM_SHARED((N_PAD, D), jnp.float32),
            pltpu.SemaphoreType.DMA,
            pltpu.SemaphoreType.DMA,
            pltpu.SemaphoreType.DMA,
            pltpu.SemaphoreType.DMA,
            pltpu.SemaphoreType.DMA,
        ],
        compiler_params=pltpu.CompilerParams(needs_layout_passes=False),
    )
    def sc_kernel(x_hbm, src_hbm, dst_hbm, sums_out, cnt_out,
                  src_v, dst_v, rows0, rows1, cnt_v, sums_sh,
                  g0, g1, s0, s1, stsem):
        cid = lax.axis_index("c")
        sid = lax.axis_index("s")
        k_t = lax.select(cid == 0, K0, K1)
        hc_base = lax.select(cid == 0, 0, K0 // HC)
        rows = (rows0, rows1)
        gsem = (g0, g1)
        ssem = (s0, s1)
        ones16 = jnp.ones((LANES,), jnp.float32)

        def stage(hc):
            off = lax.rem(hc, 2) * HC
            pltpu.sync_copy(src_hbm.at[sid, hc_base + hc],
                            src_v.at[pl.ds(off, HC)])
            pltpu.sync_copy(dst_hbm.at[sid, hc_base + hc],
                            dst_v.at[pl.ds(off, HC)])

        def issue_stage(hc):
            off = lax.rem(hc, 2) * HC
            pltpu.async_copy(src_hbm.at[sid, hc_base + hc],
                             src_v.at[pl.ds(off, HC)], stsem)
            pltpu.async_copy(dst_hbm.at[sid, hc_base + hc],
                             dst_v.at[pl.ds(off, HC)], stsem)

        def wait_stage(hc):
            off = lax.rem(hc, 2) * HC
            pltpu.make_async_copy(src_hbm.at[sid, hc_base + hc],
                                  src_v.at[pl.ds(off, HC)], stsem).wait()
            pltpu.make_async_copy(dst_hbm.at[sid, hc_base + hc],
                                  dst_v.at[pl.ds(off, HC)], stsem).wait()

        def issue_gather(j, p):
            r = lax.rem(j, 2 * HC)
            pltpu.async_copy(x_hbm.at[src_v.at[r]], rows[p], gsem[p])

        def wait_gather(j, p):
            r = lax.rem(j, 2 * HC)
            pltpu.make_async_copy(x_hbm.at[src_v.at[r]], rows[p],
                                  gsem[p]).wait()

        def issue_scatter(j, p):
            r = lax.rem(j, 2 * HC)
            pltpu.async_copy(rows[p], sums_sh.at[dst_v.at[r]], ssem[p],
                             add=True)

        def wait_scatter(j, p):
            r = lax.rem(j, 2 * HC)
            pltpu.make_async_copy(rows[p], sums_sh.at[dst_v.at[r]],
                                  ssem[p]).wait()

        def count(j):
            r = lax.rem(j, 2 * HC)

            @pl.loop(0, B, step=LANES)
            def _(t):
                plsc.addupdate_scatter(
                    cnt_v, [dst_v[r, pl.ds(t, LANES)]], ones16)

        @pl.loop(0, B)
        def _(i):
            @pl.loop(0, D, step=LANES)
            def _(j):
                rows0[i, pl.ds(j, LANES)] = jnp.zeros((LANES,), jnp.float32)

        @pl.loop(0, N_PAD, step=LANES)
        def _(i):
            cnt_v[pl.ds(i, LANES)] = jnp.zeros((LANES,), jnp.float32)

        base = sid * ROWS_PER_TILE
        for r in range(ROWS_PER_TILE // B):
            pltpu.sync_copy(rows0, sums_sh.at[pl.ds(base + r * B, B)])
        plsc.subcore_barrier()

        stage(0)
        stage(1)
        issue_gather(0, 0)

        @pl.loop(0, k_t, step=2)
        def _(j):
            wait_gather(j, 0)
            issue_scatter(j, 0)

            @pl.when(j > 0)
            def _():
                wait_scatter(j - 1, 1)

            @pl.when((lax.rem(j + 4, HC) == 0) & (j + 4 >= 2 * HC)
                     & (j + 4 < k_t))
            def _():
                issue_stage((j + 4) // HC)

            @pl.when((lax.rem(j + 1, HC) == 0) & (j + 1 >= 2 * HC))
            def _():
                wait_stage((j + 1) // HC)

            issue_gather(j + 1, 1)
            count(j)

            wait_gather(j + 1, 1)
            issue_scatter(j + 1, 1)
            wait_scatter(j, 0)

            @pl.when((lax.rem(j + 5, HC) == 0) & (j + 5 >= 2 * HC)
                     & (j + 5 < k_t))
            def _():
                issue_stage((j + 5) // HC)

            @pl.when(j + 2 < k_t)
            def _():
                @pl.when((lax.rem(j + 2, HC) == 0) & (j + 2 >= 2 * HC))
                def _():
                    wait_stage((j + 2) // HC)

                issue_gather(j + 2, 0)

            count(j + 1)

        wait_scatter(k_t - 1, 1)
        plsc.subcore_barrier()

        pltpu.sync_copy(sums_sh.at[pl.ds(base, ROWS_PER_TILE)],
                        sums_out.at[cid, pl.ds(base, ROWS_PER_TILE)])
        pltpu.sync_copy(cnt_v, cnt_out.at[cid, sid])

    return sc_kernel(x, src3, dst3)


_BLK = 1024


def _tc_body(s_ref, c_ref, x_ref, wl_ref, wr_ref, b_ref, o_ref):
    cnt = jnp.sum(c_ref[...], axis=(0, 1))[:, None]
    agg = (s_ref[0] + s_ref[1]) / jnp.maximum(cnt, 1.0)
    o_ref[...] = (
        jnp.dot(agg, wl_ref[...], preferred_element_type=jnp.float32)
        + jnp.dot(x_ref[...], wr_ref[...], preferred_element_type=jnp.float32)
        + b_ref[...]
    )


def _tc_combine(sums, cnts, x, W_l, W_r, b):
    grid = (pl.cdiv(N_NODES, _BLK),)
    return pl.pallas_call(
        _tc_body,
        grid=grid,
        in_specs=[
            pl.BlockSpec((NC, _BLK, D), lambda i: (0, i, 0)),
            pl.BlockSpec((NC, NS, _BLK), lambda i: (0, 0, i)),
            pl.BlockSpec((_BLK, D), lambda i: (i, 0)),
            pl.BlockSpec((D, D), lambda i: (0, 0)),
            pl.BlockSpec((D, D), lambda i: (0, 0)),
            pl.BlockSpec((1, D), lambda i: (0, 0)),
        ],
        out_specs=pl.BlockSpec((_BLK, D), lambda i: (i, 0)),
        out_shape=jax.ShapeDtypeStruct((N_NODES, D), jnp.float32),
    )(sums, cnts, x, W_l, W_r, b)


def kernel(x, edge_index, W_l, b_l, W_r, b_r):
    ei = edge_index.astype(jnp.int32)
    pad = E_PAD - N_EDGES
    src = jnp.concatenate([ei[0], jnp.zeros((pad,), jnp.int32)])
    dst = jnp.concatenate([ei[1], jnp.full((pad,), N_NODES, jnp.int32)])
    src3 = src.reshape(NS, NHC, HC, B)
    dst3 = dst.reshape(NS, NHC, HC, B)

    sums, cnts = _sc_segment_sum(x, src3, dst3)
    b = (b_l + b_r).reshape(1, D)
    return _tc_combine(sums, cnts, x, W_l, W_r, b)


# speedup vs baseline: 1.1013x; 1.1013x over previous
import functools

import jax
import jax.numpy as jnp
from jax import lax
from jax.experimental import pallas as pl
from jax.experimental.pallas import tpu as pltpu
from jax.experimental.pallas import tpu_sc as plsc

N_NODES = 10000
N_EDGES = 320000
D = 128

NC = 2
NS = 16
LANES = 16

B = 128
KP = 160
K0 = 120
K1 = KP - K0
HC = 8
NHC = KP // HC
E_PAD = NS * KP * B
N_PAD = 10240
ROWS_PER_TILE = N_PAD // NS


def _sc_segment_sum(x, src3, dst3):
    mesh = plsc.VectorSubcoreMesh(core_axis_name="c", subcore_axis_name="s")

    @functools.partial(
        pl.kernel,
        out_type=(
            jax.ShapeDtypeStruct((NC, N_PAD, D), jnp.float32),
            jax.ShapeDtypeStruct((NC, NS, N_PAD), jnp.float32),
        ),
        mesh=mesh,
        scratch_types=[
            pltpu.VMEM((2 * HC, B), jnp.int32),
            pltpu.VMEM((2 * HC, B), jnp.int32),
            pltpu.VMEM((B, D), jnp.float32),
            pltpu.VMEM((B, D), jnp.float32),
            pltpu.VMEM((N_PAD,), jnp.float32),
            pltpu.VMEM_SHARED((N_PAD, D), jnp.float32),
            pltpu.SemaphoreType.DMA,
            pltpu.SemaphoreType.DMA,
            pltpu.SemaphoreType.DMA,
            pltpu.SemaphoreType.DMA,
            pltpu.SemaphoreType.DMA,
        ],
        compiler_params=pltpu.CompilerParams(needs_layout_passes=False),
    )
    def sc_kernel(x_hbm, src_hbm, dst_hbm, sums_out, cnt_out,
                  src_v, dst_v, rows0, rows1, cnt_v, sums_sh,
                  g0, g1, s0, s1, stsem):
        cid = lax.axis_index("c")
        sid = lax.axis_index("s")
        k_t = lax.select(cid == 0, K0, K1)
        hc_base = lax.select(cid == 0, 0, K0 // HC)
        rows = (rows0, rows1)
        gsem = (g0, g1)
        ssem = (s0, s1)
        ones16 = jnp.ones((LANES,), jnp.float32)

        def stage(hc):
            off = lax.rem(hc, 2) * HC
            pltpu.sync_copy(src_hbm.at[sid, hc_base + hc],
                            src_v.at[pl.ds(off, HC)])
            pltpu.sync_copy(dst_hbm.at[sid, hc_base + hc],
                            dst_v.at[pl.ds(off, HC)])

        def issue_stage(hc):
            off = lax.rem(hc, 2) * HC
            pltpu.async_copy(src_hbm.at[sid, hc_base + hc],
                             src_v.at[pl.ds(off, HC)], stsem)
            pltpu.async_copy(dst_hbm.at[sid, hc_base + hc],
                             dst_v.at[pl.ds(off, HC)], stsem)

        def wait_stage(hc):
            off = lax.rem(hc, 2) * HC
            pltpu.make_async_copy(src_hbm.at[sid, hc_base + hc],
                                  src_v.at[pl.ds(off, HC)], stsem).wait()
            pltpu.make_async_copy(dst_hbm.at[sid, hc_base + hc],
                                  dst_v.at[pl.ds(off, HC)], stsem).wait()

        def issue_gather(j, p):
            r = lax.rem(j, 2 * HC)
            pltpu.async_copy(x_hbm.at[src_v.at[r]], rows[p], gsem[p])

        def wait_gather(j, p):
            r = lax.rem(j, 2 * HC)
            pltpu.make_async_copy(x_hbm.at[src_v.at[r]], rows[p],
                                  gsem[p]).wait()

        def issue_scatter(j, p):
            r = lax.rem(j, 2 * HC)
            pltpu.async_copy(rows[p], sums_sh.at[dst_v.at[r]], ssem[p],
                             add=True)

        def wait_scatter(j, p):
            r = lax.rem(j, 2 * HC)
            pltpu.make_async_copy(rows[p], sums_sh.at[dst_v.at[r]],
                                  ssem[p]).wait()

        def count(j):
            r = lax.rem(j, 2 * HC)

            @pl.loop(0, B, step=LANES)
            def _(t):
                plsc.addupdate_scatter(
                    cnt_v, [dst_v[r, pl.ds(t, LANES)]], ones16)

        @pl.loop(0, B)
        def _(i):
            @pl.loop(0, D, step=LANES)
            def _(j):
                rows0[i, pl.ds(j, LANES)] = jnp.zeros((LANES,), jnp.float32)

        @pl.loop(0, N_PAD, step=LANES)
        def _(i):
            cnt_v[pl.ds(i, LANES)] = jnp.zeros((LANES,), jnp.float32)

        base = sid * ROWS_PER_TILE
        for r in range(ROWS_PER_TILE // B):
            pltpu.sync_copy(rows0, sums_sh.at[pl.ds(base + r * B, B)])
        plsc.subcore_barrier()

        stage(0)
        stage(1)
        issue_gather(0, 0)

        @pl.loop(0, k_t, step=2)
        def _(j):
            wait_gather(j, 0)
            issue_scatter(j, 0)

            @pl.when(j > 0)
            def _():
                wait_scatter(j - 1, 1)

            @pl.when((lax.rem(j + 4, HC) == 0) & (j + 4 >= 2 * HC)
                     & (j + 4 < k_t))
            def _():
                issue_stage((j + 4) // HC)

            @pl.when((lax.rem(j + 1, HC) == 0) & (j + 1 >= 2 * HC))
            def _():
                wait_stage((j + 1) // HC)

            issue_gather(j + 1, 1)
            count(j)

            wait_gather(j + 1, 1)
            issue_scatter(j + 1, 1)
            wait_scatter(j, 0)

            @pl.when((lax.rem(j + 5, HC) == 0) & (j + 5 >= 2 * HC)
                     & (j + 5 < k_t))
            def _():
                issue_stage((j + 5) // HC)

            @pl.when(j + 2 < k_t)
            def _():
                @pl.when((lax.rem(j + 2, HC) == 0) & (j + 2 >= 2 * HC))
                def _():
                    wait_stage((j + 2) // HC)

                issue_gather(j + 2, 0)

            count(j + 1)

        wait_scatter(k_t - 1, 1)
        plsc.subcore_barrier()

        pltpu.sync_copy(sums_sh.at[pl.ds(base, ROWS_PER_TILE)],
                        sums_out.at[cid, pl.ds(base, ROWS_PER_TILE)])
        pltpu.sync_copy(cnt_v, cnt_out.at[cid, sid])

    return sc_kernel(x, src3, dst3)


_BLK = 1024


def _tc_body(s_ref, c_ref, x_ref, wl_ref, wr_ref, b_ref, o_ref):
    cnt = jnp.sum(c_ref[...], axis=(0, 1))[:, None]
    agg = (s_ref[0] + s_ref[1]) / jnp.maximum(cnt, 1.0)
    o_ref[...] = (
        jnp.dot(agg, wl_ref[...], preferred_element_type=jnp.float32)
        + jnp.dot(x_ref[...], wr_ref[...], preferred_element_type=jnp.float32)
        + b_ref[...]
    )


def _tc_combine(sums, cnts, x, W_l, W_r, b):
    grid = (pl.cdiv(N_NODES, _BLK),)
    return pl.pallas_call(
        _tc_body,
        grid=grid,
        in_specs=[
            pl.BlockSpec((NC, _BLK, D), lambda i: (0, i, 0)),
            pl.BlockSpec((NC, NS, _BLK), lambda i: (0, 0, i)),
            pl.BlockSpec((_BLK, D), lambda i: (i, 0)),
            pl.BlockSpec((D, D), lambda i: (0, 0)),
            pl.BlockSpec((D, D), lambda i: (0, 0)),
            pl.BlockSpec((1, D), lambda i: (0, 0)),
        ],
        out_specs=pl.BlockSpec((_BLK, D), lambda i: (i, 0)),
        out_shape=jax.ShapeDtypeStruct((N_NODES, D), jnp.float32),
    )(sums, cnts, x, W_l, W_r, b)


def kernel(x, edge_index, W_l, b_l, W_r, b_r):
    ei = edge_index.astype(jnp.int32)
    pad = E_PAD - N_EDGES
    src = jnp.concatenate([ei[0], jnp.zeros((pad,), jnp.int32)])
    dst = jnp.concatenate([ei[1], jnp.full((pad,), N_NODES, jnp.int32)])
    src3 = src.reshape(NS, NHC, HC, B)
    dst3 = dst.reshape(NS, NHC, HC, B)

    sums, cnts = _sc_segment_sum(x, src3, dst3)
    b = (b_l + b_r).reshape(1, D)
    return _tc_combine(sums, cnts, x, W_l, W_r, b)
